# Initial kernel scaffold; baseline (speedup 1.0000x reference)
import jax
import jax.numpy as jnp
from jax.experimental import pallas as pl


def kernel(x, edge_index, edge_attr):
    raise NotImplementedError("write your pallas kernel here")


# speedup vs baseline: 4.7711x; 4.7711x over previous
import functools

import jax
import jax.numpy as jnp
from jax import lax
from jax.experimental import pallas as pl
from jax.experimental.pallas import tpu as pltpu
from jax.experimental.pallas import tpu_sc as plsc

N_NODES = 10000
N_PAD = 10240
D_FEAT = 128
DH = 64
N_EDGES = 320000
N_TILES = 16
EDGES_PER_TILE = N_EDGES // N_TILES
W = 400
N_WIN = EDGES_PER_TILE // W
ROWS_PER_TILE = N_PAD // N_TILES
N_LAYERS = 3

_mesh = plsc.VectorSubcoreMesh(core_axis_name="c", subcore_axis_name="s")


@functools.partial(
    pl.kernel,
    out_type=(
        jax.ShapeDtypeStruct((2, N_PAD, DH), jnp.float32),
        jax.ShapeDtypeStruct((2, N_PAD, DH), jnp.float32),
        jax.ShapeDtypeStruct((2, N_PAD, DH), jnp.float32),
    ),
    scratch_types=[
        pltpu.VMEM_SHARED((N_PAD, DH), jnp.float32),
        pltpu.VMEM_SHARED((N_PAD,), jnp.float32),
        pltpu.VMEM((EDGES_PER_TILE,), jnp.float32),
        pltpu.VMEM((W,), jnp.float32),
        pltpu.VMEM((W,), jnp.float32),
        pltpu.VMEM((W,), jnp.float32),
        pltpu.VMEM((W,), jnp.float32),
        pltpu.VMEM((W,), jnp.int32),
        pltpu.VMEM((W,), jnp.int32),
        pltpu.VMEM((W, DH), jnp.float32),
        pltpu.VMEM((ROWS_PER_TILE // 2, DH), jnp.float32),
        pltpu.VMEM((ROWS_PER_TILE,), jnp.float32),
        pltpu.VMEM((64, DH), jnp.float32),
        pltpu.VMEM((W,), jnp.float32),
        pltpu.SemaphoreType.DMA,
    ],
    mesh=_mesh,
    compiler_params=pltpu.CompilerParams(use_tc_tiling_on_sc=False),
)
def _diffuse(xin, src_h, dst_h, eat_h, out_h, bufa_h, bufb_h,
             agg, deg, w_all, wb0, wb1, wb2, wb3, srcw, dstw, rows, nbuf, invb,
             zrows, onesb, sem):
    c = lax.axis_index("c")
    s = lax.axis_index("s")
    ebase = s * EDGES_PER_TILE
    row0 = s * ROWS_PER_TILE

    zero16 = jnp.zeros((16,), jnp.float32)
    one16 = jnp.ones((16,), jnp.float32)

    def z_body(i, _):
        for m in range(DH // 16):
            zrows[i, pl.ds(m * 16, 16)] = zero16
        return 0
    lax.fori_loop(0, 64, z_body, 0)

    def ones_body(j, _):
        onesb[pl.ds(j * 16, 16)] = one16
        return 0
    lax.fori_loop(0, W // 16, ones_body, 0)

    def invz_body(j, _):
        invb[pl.ds(j * 16, 16)] = zero16
        return 0
    lax.fori_loop(0, ROWS_PER_TILE // 16, invz_body, 0)

    for b in range(ROWS_PER_TILE // 64):
        pltpu.sync_copy(zrows, agg.at[pl.ds(row0 + b * 64, 64)])
    pltpu.sync_copy(invb, deg.at[pl.ds(row0, ROWS_PER_TILE)])

    def w_win(k, _):
        base = ebase + k * W
        for r, wb in enumerate((wb0, wb1, wb2, wb3)):
            pltpu.sync_copy(eat_h.at[pl.ds(r * N_EDGES + base, W)], wb)
        def w_body(j, _):
            sl = pl.ds(j * 16, 16)
            v = (wb0[sl] + wb1[sl]) + (wb2[sl] + wb3[sl])
            w_all[pl.ds(k * W + j * 16, 16)] = v * 0.25
            return 0
        lax.fori_loop(0, W // 16, w_body, 0)
        return 0
    lax.fori_loop(0, N_WIN, w_win, 0)

    plsc.subcore_barrier()

    def deg_win(k, _):
        pltpu.sync_copy(dst_h.at[pl.ds(ebase + k * W, W)], dstw)
        pltpu.sync_copy(onesb, deg.at[dstw], add=True)
        return 0
    lax.fori_loop(0, N_WIN, deg_win, 0)

    plsc.subcore_barrier()

    pltpu.sync_copy(deg.at[pl.ds(row0, ROWS_PER_TILE)], invb)
    def inv_body(j, _):
        sl = pl.ds(j * 16, 16)
        invb[sl] = one16 / jnp.maximum(invb[sl], one16)
        return 0
    lax.fori_loop(0, ROWS_PER_TILE // 16, inv_body, 0)

    plsc.subcore_barrier()

    def run_layer(x_ref, y_ref):
        def e_win(k, _):
            base = ebase + k * W
            pltpu.sync_copy(src_h.at[pl.ds(base, W)], srcw)
            pltpu.sync_copy(dst_h.at[pl.ds(base, W)], dstw)
            pltpu.async_copy(x_ref.at[srcw], rows, sem).wait()
            def scale_body(jo, _):
                w16 = w_all[pl.ds(k * W + jo * 16, 16)]
                for r in range(16):
                    wj = jnp.full((16,), w16[r], jnp.float32)
                    j = jo * 16 + r
                    for m in range(DH // 16):
                        sl = pl.ds(m * 16, 16)
                        rows[j, sl] = rows[j, sl] * wj
                return 0
            lax.fori_loop(0, W // 16, scale_body, 0)
            pltpu.sync_copy(rows, agg.at[dstw], add=True)
            return 0
        lax.fori_loop(0, N_WIN, e_win, 0)

        plsc.subcore_barrier()

        half = ROWS_PER_TILE // 2
        for ch in range(2):
            rbase = row0 + ch * half
            pltpu.sync_copy(agg.at[pl.ds(rbase, half)], nbuf)
            def n_body(jo, _):
                iv16 = invb[pl.ds(ch * half + jo * 16, 16)]
                for r in range(16):
                    iv = jnp.full((16,), iv16[r], jnp.float32)
                    j = jo * 16 + r
                    for m in range(DH // 16):
                        sl = pl.ds(m * 16, 16)
                        nbuf[j, sl] = nbuf[j, sl] * iv
                return 0
            lax.fori_loop(0, half // 16, n_body, 0)
            pltpu.sync_copy(nbuf, y_ref.at[pl.ds(rbase, half)])

        for b in range(ROWS_PER_TILE // 64):
            pltpu.sync_copy(zrows, agg.at[pl.ds(row0 + b * 64, 64)])

        plsc.subcore_barrier()

    run_layer(xin.at[c], bufa_h.at[c])
    run_layer(bufa_h.at[c], bufb_h.at[c])
    run_layer(bufb_h.at[c], out_h.at[c])


def kernel(x, edge_index, edge_attr):
    src = edge_index[0].astype(jnp.int32)
    dst = edge_index[1].astype(jnp.int32)
    eat = edge_attr.T.reshape(-1)
    xin = jnp.stack([x[:, :DH], x[:, DH:]])
    out, _, _ = _diffuse(xin, src, dst, eat)
    return jnp.concatenate([out[0, :N_NODES], out[1, :N_NODES]], axis=1)
